# baseline (speedup 1.0000x reference)
import jax
import jax.numpy as jnp
from jax import lax
from jax.experimental import pallas as pl
from jax.experimental.pallas import tpu as pltpu

_DIM = 768
_ML = 1000
_S = 512
_BJ = 8
_NST = _S // _BJ
_DEPTH = 4


def _body(r_hbm, out_hbm, w, g, load_sem, sems):
    jb = pl.program_id(0)

    @pl.when(jb == 0)
    def _stage():
        cp = pltpu.make_async_copy(r_hbm.at[pl.ds(488, 1032), :], w, load_sem)
        cp.start()
        cp.wait()

        def _chunk(k, carry):
            base = pl.multiple_of(8 * k, 8)
            v = w[pl.ds(base, 16), :]
            for d in range(_BJ):
                g[pl.ds(base, 8), d * _DIM:(d + 1) * _DIM] = v[8 - d:16 - d, :]
            return carry

        lax.fori_loop(0, 1024 // 8, _chunk, 0)

    def _copy(step, slot):
        off = pl.multiple_of(504 - _BJ * step, 8)
        return pltpu.make_async_copy(
            g.at[pl.ds(off, _S), :],
            out_hbm.at[:, pl.ds(step * _BJ * _DIM, _BJ * _DIM)],
            sems.at[slot],
        )

    _copy(jb, jb % _DEPTH).start()

    @pl.when(jb >= _DEPTH - 1)
    def _drain_prev():
        _copy(jb + 1 - _DEPTH, (jb + 1) % _DEPTH).wait()

    @pl.when(jb == _NST - 1)
    def _drain_last():
        for t in range(1, _DEPTH):
            _copy(jb + 1 - _DEPTH + t, (jb + 1 + t) % _DEPTH).wait()


def kernel(hidden_states, pe_k_weight):
    s = hidden_states.shape[1]
    out2d = pl.pallas_call(
        _body,
        grid=(_NST,),
        in_specs=[pl.BlockSpec(memory_space=pl.ANY)],
        out_specs=pl.BlockSpec(memory_space=pl.ANY),
        out_shape=jax.ShapeDtypeStruct((s, s * _DIM), jnp.float32),
        scratch_shapes=[
            pltpu.VMEM((1032, _DIM), jnp.float32),
            pltpu.VMEM((1024, _BJ * _DIM), jnp.float32),
            pltpu.SemaphoreType.DMA,
            pltpu.SemaphoreType.DMA((_DEPTH,)),
        ],
    )(pe_k_weight)
    return out2d.reshape(s, s, _DIM)


# speedup vs baseline: 1.3083x; 1.3083x over previous
import jax
import jax.numpy as jnp
from jax import lax
from jax.experimental import pallas as pl
from jax.experimental.pallas import tpu as pltpu
from jax.experimental.pallas import tpu_sc as plsc

_DIM = 768
_ML = 1000
_S = 512
_NC = 2
_NW = 32
_IPW = _S // _NW
_CH = 64
_CPR = _S // _CH
_NCK = _IPW * _CPR


def _coords(c, i0):
    return i0 + c // _CPR, (c % _CPR) * _CH


def _sc_body(tab_hbm, out_hbm, idx, buf, gsem, ssem):
    wid = lax.axis_index("s") * _NC + lax.axis_index("c")
    i0 = wid * _IPW

    def _fill_idx(c, slot):
        i, j0 = _coords(c, i0)
        base = i + _ML - j0
        for k in range(_CH // 16):
            idx[slot, pl.ds(16 * k, 16)] = base - 16 * k - lax.iota(jnp.int32, 16)

    def _gather(slot):
        return pltpu.make_async_copy(
            tab_hbm.at[idx.at[slot]], buf.at[slot], gsem.at[slot]
        )

    def _scatter(c, slot):
        i, j0 = _coords(c, i0)
        return pltpu.make_async_copy(
            buf.at[slot], out_hbm.at[i, pl.ds(j0, _CH), :], ssem.at[slot]
        )

    _fill_idx(0, 0)
    _gather(0).start()

    def _step(c, carry):
        slot = c % 2
        prev = 1 - slot
        _gather(prev).wait()
        _scatter(c - 1, prev).start()

        @pl.when(c >= 2)
        def _():
            _scatter(0, slot).wait()

        _fill_idx(c, slot)
        _gather(slot).start()
        return carry

    lax.fori_loop(1, _NCK, _step, 0, unroll=False)

    last = _NCK - 1
    slot = last % 2
    _gather(slot).wait()
    _scatter(last, slot).start()
    _scatter(0, 1 - slot).wait()
    _scatter(0, slot).wait()


def kernel(hidden_states, pe_k_weight):
    s = hidden_states.shape[1]
    mesh = plsc.VectorSubcoreMesh(core_axis_name="c", subcore_axis_name="s")
    run = pl.kernel(
        _sc_body,
        out_type=jax.ShapeDtypeStruct((s, s, _DIM), jnp.float32),
        mesh=mesh,
        scratch_types=[
            pltpu.VMEM((2, _CH), jnp.int32),
            pltpu.VMEM((2, _CH, _DIM), jnp.float32),
            pltpu.SemaphoreType.DMA((2,)),
            pltpu.SemaphoreType.DMA((2,)),
        ],
    )
    return run(pe_k_weight)
